# baseline (speedup 1.0000x reference)
import functools

import jax
import jax.numpy as jnp
from jax import lax
from jax.experimental import pallas as pl
from jax.experimental.pallas import tpu as pltpu
from jax.experimental.pallas import tpu_sc as plsc

R = 128
N = 100000
NC = 2
NS = 16
SEG = 6272
NPAD = NS * SEG
VPS = SEG // 16
CH = SEG // 128
LAST = N - (NS - 1) * SEG
VBITS = 17
SHIFTS = (0, 11, 22)
NBINS = (2048, 2048, 1024)
NPASS = 3
ROWS_PER_SC = R // NC


def _keys_body(p_ref, n_ref, o_ref):
    x = p_ref[...] + n_ref[...]
    u = lax.bitcast_convert_type(x, jnp.int32)
    o_ref[...] = u ^ jnp.where(u < 0, jnp.int32(-1), jnp.int32(-(2**31)))


def _make_keys(policy, noise):
    return pl.pallas_call(
        _keys_body,
        grid=(16,),
        in_specs=[pl.BlockSpec((8, N), lambda i: (i, 0)),
                  pl.BlockSpec((8, N), lambda i: (i, 0))],
        out_specs=pl.BlockSpec((8, N), lambda i: (i, 0)),
        out_shape=jax.ShapeDtypeStruct((R, N), jnp.int32),
    )(policy, noise)


def _sort_body(keys_hbm, out_hbm, keys_l, vals_l, pack_l, digs_l, r1_l,
               vals0_l, dest_l, hist_l, colblk_l, pref_l, tot_l, tots_l,
               prow_l, base_l, vsum_l, carr_l, ska, sva, skb, shist, spref,
               stot, sem):
    c = lax.axis_index("c")
    s = lax.axis_index("s")
    iota16 = lax.iota(jnp.int32, 16)
    zeros16 = jnp.zeros((16,), jnp.int32)

    def init_body(v, carry):
        vals0_l[pl.ds(v * 16, 16)] = iota16 + (s * SEG + v * 16)
        return carry
    lax.fori_loop(0, VPS, init_body, 0)

    def row_body(i, carry):
        r = i * NC + c
        row_off = r * N

        for p in range(NPASS):
            shift = SHIFTS[p]
            nb = NBINS[p]
            own = 128
            npart = nb // own

            if p == 0:
                @pl.when(s < NS - 1)
                def _():
                    pltpu.sync_copy(
                        keys_hbm.at[pl.ds(row_off + s * SEG, SEG)], keys_l)

                @pl.when(s == NS - 1)
                def _():
                    pltpu.sync_copy(
                        keys_hbm.at[pl.ds(row_off + s * SEG, LAST)],
                        keys_l.at[pl.ds(0, LAST)])
                    def pad_body(v, carry2):
                        keys_l[pl.ds(LAST + v * 16, 16)] = (
                            zeros16 - 1)
                        return carry2
                    lax.fori_loop(0, (SEG - LAST) // 16, pad_body, 0)
            elif p == 1:
                pltpu.sync_copy(ska.at[pl.ds(s * SEG, SEG)], keys_l)
                pltpu.sync_copy(sva.at[pl.ds(s * SEG, SEG)], vals_l)
            else:
                pltpu.sync_copy(skb.at[pl.ds(s * SEG, SEG)], keys_l)

            def zero_body(j, carry2):
                hist_l[pl.ds(j * 16, 16)] = zeros16
                return carry2
            lax.fori_loop(0, nb // 16, zero_body, 0, unroll=4)

            def sweep1(v, carry2):
                off = v * 16
                k = keys_l[pl.ds(off, 16)]
                if p == 0:
                    d = k & (nb - 1)
                elif p == 1:
                    d = lax.shift_right_logical(k, shift) & (nb - 1)
                    pack_l[pl.ds(off, 16)] = (
                        lax.shift_left(lax.shift_right_logical(k, 22), VBITS)
                        | vals_l[pl.ds(off, 16)])
                else:
                    d = lax.shift_right_logical(k, VBITS)
                    vals_l[pl.ds(off, 16)] = k & ((1 << VBITS) - 1)
                cnt, last = plsc.scan_count(d)
                h = plsc.load_gather(hist_l, [d])
                digs_l[pl.ds(off, 16)] = d
                r1_l[pl.ds(off, 16)] = h + cnt - 1
                plsc.addupdate_scatter(hist_l, [d], cnt, mask=last)
                return carry2
            lax.fori_loop(0, VPS, sweep1, 0, unroll=2)

            pltpu.sync_copy(hist_l.at[pl.ds(0, nb)], shist.at[s, pl.ds(0, nb)])
            plsc.subcore_barrier()

            @pl.when(s < npart)
            def _():
                boff = pl.multiple_of(s * own, own)
                pltpu.sync_copy(shist.at[:, pl.ds(boff, own)], colblk_l)
                for j in range(own // 16):
                    run = zeros16
                    for t2 in range(NS):
                        v2 = colblk_l[t2, pl.ds(j * 16, 16)]
                        pref_l[t2, pl.ds(j * 16, 16)] = run
                        run = run + v2
                    tot_l[pl.ds(j * 16, 16)] = run
                pdescs = [
                    pltpu.async_copy(pref_l, spref.at[:, pl.ds(boff, own)],
                                     sem),
                    pltpu.async_copy(tot_l, stot.at[pl.ds(boff, own)], sem),
                ]
                for dsc in pdescs:
                    dsc.wait()
            plsc.subcore_barrier()

            pltpu.sync_copy(stot.at[pl.ds(0, nb)], tots_l.at[pl.ds(0, nb)])
            pltpu.sync_copy(spref.at[s, pl.ds(0, nb)], prow_l.at[pl.ds(0, nb)])

            nv = nb // 16
            for m in range(nv // 16):
                vs = zeros16
                bidx = (iota16 + m * 16) * 16
                for l in range(16):
                    vs = vs + plsc.load_gather(tots_l, [bidx + l])
                vsum_l[pl.ds(m * 16, 16)] = vs

            def vscan(m, carry2):
                t16 = vsum_l[pl.ds(m * 16, 16)]
                carr_l[pl.ds(m * 16, 16)] = carry2 + plsc.cumsum(t16) - t16
                return carry2 + jnp.sum(t16)
            lax.fori_loop(0, nv // 16, vscan, 0)

            def scan_body(j, carry2):
                t16 = tots_l[pl.ds(j * 16, 16)]
                cj = plsc.load_gather(carr_l, [zeros16 + j])
                base_l[pl.ds(j * 16, 16)] = (
                    cj + plsc.cumsum(t16) - t16 + prow_l[pl.ds(j * 16, 16)])
                return carry2
            lax.fori_loop(0, nv, scan_body, 0, unroll=4)

            if p == 0:
                srcs = (keys_l, ska), (vals0_l, sva)
            elif p == 1:
                srcs = ((pack_l, skb),)
            else:
                srcs = ((vals_l, sva),)

            def sweep2(j, carry2):
                for q in range(8):
                    off = j * 128 + q * 16
                    d = digs_l[pl.ds(off, 16)]
                    dest = plsc.load_gather(base_l, [d]) + r1_l[pl.ds(off, 16)]
                    dest_l[j, pl.ds(q * 16, 16)] = dest
                for src, dst in srcs:
                    pltpu.async_copy(src.at[pl.ds(j * 128, 128)],
                                     dst.at[dest_l.at[j]], sem)
                return carry2
            lax.fori_loop(0, CH, sweep2, 0)
            for src, _ in srcs:
                pltpu.make_async_copy(
                    keys_hbm.at[pl.ds(0, SEG)], src, sem).wait()
            plsc.subcore_barrier()

        pltpu.sync_copy(sva.at[pl.ds(s * SEG, SEG)], vals_l)

        @pl.when(s < NS - 1)
        def _():
            pltpu.sync_copy(vals_l,
                            out_hbm.at[pl.ds(row_off + s * SEG, SEG)])

        @pl.when(s == NS - 1)
        def _():
            pltpu.sync_copy(vals_l.at[pl.ds(0, LAST)],
                            out_hbm.at[pl.ds(row_off + s * SEG, LAST)])
        plsc.subcore_barrier()
        return carry
    lax.fori_loop(0, ROWS_PER_SC, row_body, 0)


def _sort(keys):
    mesh = plsc.VectorSubcoreMesh(core_axis_name="c", subcore_axis_name="s")
    f = functools.partial(
        pl.kernel,
        out_type=jax.ShapeDtypeStruct((R * N,), jnp.int32),
        mesh=mesh,
        compiler_params=pltpu.CompilerParams(needs_layout_passes=False),
        scratch_types=[
            pltpu.VMEM((SEG,), jnp.int32),
            pltpu.VMEM((SEG,), jnp.int32),
            pltpu.VMEM((SEG,), jnp.int32),
            pltpu.VMEM((SEG,), jnp.int32),
            pltpu.VMEM((SEG,), jnp.int32),
            pltpu.VMEM((SEG,), jnp.int32),
            pltpu.VMEM((CH, 128), jnp.int32),
            pltpu.VMEM((2048,), jnp.int32),
            pltpu.VMEM((NS, 128), jnp.int32),
            pltpu.VMEM((NS, 128), jnp.int32),
            pltpu.VMEM((128,), jnp.int32),
            pltpu.VMEM((2048,), jnp.int32),
            pltpu.VMEM((2048,), jnp.int32),
            pltpu.VMEM((2048,), jnp.int32),
            pltpu.VMEM((128,), jnp.int32),
            pltpu.VMEM((128,), jnp.int32),
            pltpu.VMEM_SHARED((NPAD,), jnp.int32),
            pltpu.VMEM_SHARED((NPAD,), jnp.int32),
            pltpu.VMEM_SHARED((NPAD,), jnp.int32),
            pltpu.VMEM_SHARED((NS, 2048), jnp.int32),
            pltpu.VMEM_SHARED((NS, 2048), jnp.int32),
            pltpu.VMEM_SHARED((2048,), jnp.int32),
            pltpu.SemaphoreType.DMA,
        ],
    )(_sort_body)
    return f(keys.reshape(R * N)).reshape(R, N)


def kernel(policy):
    nkey = jax.random.fold_in(jax.random.key(0), 1)
    u = jax.random.uniform(nkey, jnp.shape(policy), dtype=policy.dtype,
                           minval=1e-20, maxval=1.0)
    noise = -jnp.log(-jnp.log(u))
    keys = _make_keys(policy, noise)
    return _sort(keys)


# speedup vs baseline: 1.0692x; 1.0692x over previous
import functools

import jax
import jax.numpy as jnp
from jax import lax
from jax.experimental import pallas as pl
from jax.experimental.pallas import tpu as pltpu
from jax.experimental.pallas import tpu_sc as plsc

R = 128
N = 100000
NC = 2
NS = 16
SEG = 6272
NPAD = NS * SEG
VPS = SEG // 16
CH = SEG // 128
LAST = N - (NS - 1) * SEG
VBITS = 17
SHIFTS = (0, 11, 22)
NBINS = (2048, 2048, 1024)
NPASS = 3
ROWS_PER_SC = R // NC


def _keys_body(p_ref, n_ref, o_ref):
    x = p_ref[...] + n_ref[...]
    u = lax.bitcast_convert_type(x, jnp.int32)
    o_ref[...] = u ^ jnp.where(u < 0, jnp.int32(-1), jnp.int32(-(2**31)))


def _make_keys(policy, noise):
    return pl.pallas_call(
        _keys_body,
        grid=(16,),
        in_specs=[pl.BlockSpec((8, N), lambda i: (i, 0)),
                  pl.BlockSpec((8, N), lambda i: (i, 0))],
        out_specs=pl.BlockSpec((8, N), lambda i: (i, 0)),
        out_shape=jax.ShapeDtypeStruct((R, N), jnp.int32),
    )(policy, noise)


def _sort_body(keys_hbm, out_hbm, keys_l, vals_l, pack_l, digs_l, r1_l,
               vals0_l, dest_l, hist_l, colblk_l, pref_l, tot_l, tots_l,
               prow_l, base_l, ska, sva, skb, shist, spref, stot, sem):
    c = lax.axis_index("c")
    s = lax.axis_index("s")
    iota16 = lax.iota(jnp.int32, 16)
    zeros16 = jnp.zeros((16,), jnp.int32)

    def init_body(v, carry):
        vals0_l[pl.ds(v * 16, 16)] = iota16 + (s * SEG + v * 16)
        return carry
    lax.fori_loop(0, VPS, init_body, 0)

    def row_body(i, carry):
        r = i * NC + c
        row_off = r * N

        for p in range(NPASS):
            shift = SHIFTS[p]
            nb = NBINS[p]
            own = 128
            npart = nb // own

            if p == 0:
                @pl.when(s < NS - 1)
                def _():
                    pltpu.sync_copy(
                        keys_hbm.at[pl.ds(row_off + s * SEG, SEG)], keys_l)

                @pl.when(s == NS - 1)
                def _():
                    pltpu.sync_copy(
                        keys_hbm.at[pl.ds(row_off + s * SEG, LAST)],
                        keys_l.at[pl.ds(0, LAST)])
                    def pad_body(v, carry2):
                        keys_l[pl.ds(LAST + v * 16, 16)] = (
                            zeros16 - 1)
                        return carry2
                    lax.fori_loop(0, (SEG - LAST) // 16, pad_body, 0)
            elif p == 1:
                pltpu.sync_copy(ska.at[pl.ds(s * SEG, SEG)], keys_l)
                pltpu.sync_copy(sva.at[pl.ds(s * SEG, SEG)], vals_l)
            else:
                pltpu.sync_copy(skb.at[pl.ds(s * SEG, SEG)], keys_l)

            def zero_body(j, carry2):
                hist_l[pl.ds(j * 16, 16)] = zeros16
                return carry2
            lax.fori_loop(0, nb // 16, zero_body, 0, unroll=4)

            def sweep1(v, carry2):
                off = v * 16
                k = keys_l[pl.ds(off, 16)]
                if p == 0:
                    d = k & (nb - 1)
                elif p == 1:
                    d = lax.shift_right_logical(k, shift) & (nb - 1)
                    pack_l[pl.ds(off, 16)] = (
                        lax.shift_left(lax.shift_right_logical(k, 22), VBITS)
                        | vals_l[pl.ds(off, 16)])
                else:
                    d = lax.shift_right_logical(k, VBITS)
                    vals_l[pl.ds(off, 16)] = k & ((1 << VBITS) - 1)
                cnt, last = plsc.scan_count(d)
                h = plsc.load_gather(hist_l, [d])
                digs_l[pl.ds(off, 16)] = d
                r1_l[pl.ds(off, 16)] = h + cnt - 1
                plsc.addupdate_scatter(hist_l, [d], cnt, mask=last)
                return carry2
            lax.fori_loop(0, VPS, sweep1, 0)

            pltpu.sync_copy(hist_l.at[pl.ds(0, nb)], shist.at[s, pl.ds(0, nb)])
            plsc.subcore_barrier()

            @pl.when(s < npart)
            def _():
                boff = pl.multiple_of(s * own, own)
                pltpu.sync_copy(shist.at[:, pl.ds(boff, own)], colblk_l)
                for j in range(own // 16):
                    run = zeros16
                    for t2 in range(NS):
                        v2 = colblk_l[t2, pl.ds(j * 16, 16)]
                        pref_l[t2, pl.ds(j * 16, 16)] = run
                        run = run + v2
                    tot_l[pl.ds(j * 16, 16)] = run
                pdescs = [
                    pltpu.async_copy(pref_l, spref.at[:, pl.ds(boff, own)],
                                     sem),
                    pltpu.async_copy(tot_l, stot.at[pl.ds(boff, own)], sem),
                ]
                for dsc in pdescs:
                    dsc.wait()
            plsc.subcore_barrier()

            pltpu.sync_copy(stot.at[pl.ds(0, nb)], tots_l.at[pl.ds(0, nb)])
            pltpu.sync_copy(spref.at[s, pl.ds(0, nb)], prow_l.at[pl.ds(0, nb)])

            def scan_body(j, carry2):
                t16 = tots_l[pl.ds(j * 16, 16)]
                ex = carry2 + plsc.cumsum(t16) - t16
                base_l[pl.ds(j * 16, 16)] = ex + prow_l[pl.ds(j * 16, 16)]
                return carry2 + jnp.sum(t16)
            lax.fori_loop(0, nb // 16, scan_body, 0)

            if p == 0:
                srcs = (keys_l, ska), (vals0_l, sva)
            elif p == 1:
                srcs = ((pack_l, skb),)
            else:
                srcs = ((vals_l, sva),)

            def sweep2(j, carry2):
                for q in range(8):
                    off = j * 128 + q * 16
                    d = digs_l[pl.ds(off, 16)]
                    dest = plsc.load_gather(base_l, [d]) + r1_l[pl.ds(off, 16)]
                    dest_l[j, pl.ds(q * 16, 16)] = dest
                for src, dst in srcs:
                    pltpu.async_copy(src.at[pl.ds(j * 128, 128)],
                                     dst.at[dest_l.at[j]], sem)
                return carry2
            lax.fori_loop(0, CH, sweep2, 0)
            for src, _ in srcs:
                pltpu.make_async_copy(
                    keys_hbm.at[pl.ds(0, SEG)], src, sem).wait()
            plsc.subcore_barrier()

        pltpu.sync_copy(sva.at[pl.ds(s * SEG, SEG)], vals_l)

        @pl.when(s < NS - 1)
        def _():
            pltpu.sync_copy(vals_l,
                            out_hbm.at[pl.ds(row_off + s * SEG, SEG)])

        @pl.when(s == NS - 1)
        def _():
            pltpu.sync_copy(vals_l.at[pl.ds(0, LAST)],
                            out_hbm.at[pl.ds(row_off + s * SEG, LAST)])
        plsc.subcore_barrier()
        return carry
    lax.fori_loop(0, ROWS_PER_SC, row_body, 0)


def _sort(keys):
    mesh = plsc.VectorSubcoreMesh(core_axis_name="c", subcore_axis_name="s")
    f = functools.partial(
        pl.kernel,
        out_type=jax.ShapeDtypeStruct((R * N,), jnp.int32),
        mesh=mesh,
        compiler_params=pltpu.CompilerParams(needs_layout_passes=False),
        scratch_types=[
            pltpu.VMEM((SEG,), jnp.int32),
            pltpu.VMEM((SEG,), jnp.int32),
            pltpu.VMEM((SEG,), jnp.int32),
            pltpu.VMEM((SEG,), jnp.int32),
            pltpu.VMEM((SEG,), jnp.int32),
            pltpu.VMEM((SEG,), jnp.int32),
            pltpu.VMEM((CH, 128), jnp.int32),
            pltpu.VMEM((2048,), jnp.int32),
            pltpu.VMEM((NS, 128), jnp.int32),
            pltpu.VMEM((NS, 128), jnp.int32),
            pltpu.VMEM((128,), jnp.int32),
            pltpu.VMEM((2048,), jnp.int32),
            pltpu.VMEM((2048,), jnp.int32),
            pltpu.VMEM((2048,), jnp.int32),
            pltpu.VMEM_SHARED((NPAD,), jnp.int32),
            pltpu.VMEM_SHARED((NPAD,), jnp.int32),
            pltpu.VMEM_SHARED((NPAD,), jnp.int32),
            pltpu.VMEM_SHARED((NS, 2048), jnp.int32),
            pltpu.VMEM_SHARED((NS, 2048), jnp.int32),
            pltpu.VMEM_SHARED((2048,), jnp.int32),
            pltpu.SemaphoreType.DMA,
        ],
    )(_sort_body)
    return f(keys.reshape(R * N)).reshape(R, N)


def kernel(policy):
    nkey = jax.random.fold_in(jax.random.key(0), 1)
    u = jax.random.uniform(nkey, jnp.shape(policy), dtype=policy.dtype,
                           minval=1e-20, maxval=1.0)
    noise = -jnp.log(-jnp.log(u))
    keys = _make_keys(policy, noise)
    return _sort(keys)
